# Initial kernel scaffold; baseline (speedup 1.0000x reference)
import jax
import jax.numpy as jnp
from jax.experimental import pallas as pl


def kernel(x, edge_index, batch, w_mul, W1, b1, W2, b2, W3, b3):
    raise NotImplementedError("write your pallas kernel here")


# speedup vs baseline: 3.5605x; 3.5605x over previous
import functools

import jax
import jax.numpy as jnp
from jax import lax
from jax.experimental import pallas as pl
from jax.experimental.pallas import tpu as pltpu
from jax.experimental.pallas import tpu_sc as plsc

_NC = 2
_NS = 16
_L = 16

_GB = 1024
_IDXW = 128
_G = 8


def _splat(vec16, e):
  dnums = lax.GatherDimensionNumbers(
      offset_dims=(), collapsed_slice_dims=(0,), start_index_map=(0,))
  idx = jnp.full((_L, 1), e, dtype=jnp.int32)
  return lax.gather(vec16, idx, dnums, (1,),
                    mode=lax.GatherScatterMode.PROMISE_IN_BOUNDS)


def _sc_aggregate(h, src2, dst2, w, n_nodes, n_feat, ep):
  assert n_feat % _L == 0
  ew = ep // (_NC * _NS)
  nb = ew // _GB
  assert ew % _GB == 0
  rows_ps = n_nodes // _NS
  assert n_nodes % _NS == 0
  nj = _GB // _IDXW

  mesh = plsc.VectorSubcoreMesh(core_axis_name="c", subcore_axis_name="s")

  @functools.partial(
      pl.kernel,
      mesh=mesh,
      compiler_params=pltpu.CompilerParams(use_tc_tiling_on_sc=False),
      out_type=jax.ShapeDtypeStruct((_NC, n_nodes, n_feat), jnp.float32),
      scratch_types=[
          pltpu.VMEM((nj, _IDXW), jnp.int32),
          pltpu.VMEM((nj, _IDXW), jnp.int32),
          pltpu.VMEM((_GB,), jnp.float32),
          pltpu.VMEM((_GB, n_feat), jnp.float32),
          pltpu.VMEM_SHARED((n_nodes, n_feat), jnp.float32),
          pltpu.SemaphoreType.DMA,
      ],
  )
  def agg(h_hbm, src_hbm, dst_hbm, w_hbm, out_hbm,
          idx_v, dstidx_v, w_v, rows_v, acc_sh, sem):
    cid = lax.axis_index("c")
    sid = lax.axis_index("s")
    wid = cid * _NS + sid

    def zero_row(i, _):
      for k in range(n_feat // _L):
        rows_v[i, pl.ds(k * _L, _L)] = jnp.zeros((_L,), jnp.float32)
      return _
    lax.fori_loop(0, _GB, zero_row, None)
    r0 = sid * rows_ps
    pltpu.sync_copy(rows_v.at[pl.ds(0, rows_ps)], acc_sh.at[pl.ds(r0, rows_ps)])
    plsc.subcore_barrier()

    def block(b, _):
      roff = (wid * nb + b) * nj
      eoff = (wid * nb + b) * _GB
      pltpu.sync_copy(src_hbm.at[pl.ds(roff, nj)], idx_v)
      pltpu.sync_copy(dst_hbm.at[pl.ds(roff, nj)], dstidx_v)
      pltpu.sync_copy(w_hbm.at[pl.ds(eoff, _GB)], w_v)
      copies = [
          pltpu.async_copy(h_hbm.at[idx_v.at[j]],
                           rows_v.at[pl.ds(j * _IDXW, _IDXW)], sem)
          for j in range(nj)
      ]
      for cpy in copies:
        cpy.wait()
      def scale_group(g, _):
        wv = w_v[pl.ds(g * _L, _L)]
        for e in range(_L):
          we = _splat(wv, e)
          r = g * _L + e
          for k in range(n_feat // _L):
            rows_v[r, pl.ds(k * _L, _L)] = rows_v[r, pl.ds(k * _L, _L)] * we
        return _
      lax.fori_loop(0, _GB // _L, scale_group, None)
      for j in range(nj):
        pltpu.sync_copy(rows_v.at[pl.ds(j * _IDXW, _IDXW)],
                        acc_sh.at[dstidx_v.at[j]], add=True)
      return _
    lax.fori_loop(0, nb, block, None)

    plsc.subcore_barrier()
    pltpu.sync_copy(acc_sh.at[pl.ds(r0, rows_ps)],
                    out_hbm.at[cid, pl.ds(r0, rows_ps)])

  return agg(h, src2, dst2, w)


def _mm_bias(x, W, b, n_out):
  f = x.shape[1]
  h = W.shape[1]
  n = n_out
  br = 1024
  grid = (n + br - 1) // br

  def body(x_ref, w_ref, b_ref, o_ref):
    o_ref[...] = (
        jnp.dot(x_ref[...], w_ref[...], preferred_element_type=jnp.float32)
        + b_ref[...])

  return pl.pallas_call(
      body,
      grid=(grid,),
      in_specs=[
          pl.BlockSpec((br, f), lambda i: (i, 0)),
          pl.BlockSpec((f, h), lambda i: (0, 0)),
          pl.BlockSpec((1, h), lambda i: (0, 0)),
      ],
      out_specs=pl.BlockSpec((br, h), lambda i: (i, 0)),
      out_shape=jax.ShapeDtypeStruct((n, h), jnp.float32),
  )(x, W, b.reshape(1, h))


def _sum_relu_mm_bias(p0, p1, W, b):
  n, f = p0.shape
  h = W.shape[1]
  br = 1024
  grid = (n + br - 1) // br

  def body(a_ref, c_ref, w_ref, b_ref, o_ref):
    s = jnp.maximum(a_ref[...] + c_ref[...], 0.0)
    o_ref[...] = (
        jnp.dot(s, w_ref[...], preferred_element_type=jnp.float32)
        + b_ref[...])

  return pl.pallas_call(
      body,
      grid=(grid,),
      in_specs=[
          pl.BlockSpec((br, f), lambda i: (i, 0)),
          pl.BlockSpec((br, f), lambda i: (i, 0)),
          pl.BlockSpec((f, h), lambda i: (0, 0)),
          pl.BlockSpec((1, h), lambda i: (0, 0)),
      ],
      out_specs=pl.BlockSpec((br, h), lambda i: (i, 0)),
      out_shape=jax.ShapeDtypeStruct((n, h), jnp.float32),
  )(p0, p1, W, b.reshape(1, h))


def _pool_head(p0, p1, batch2, W3, b3):
  n, h = p0.shape
  c = W3.shape[1]

  def body(a_ref, c_ref, bat_ref, w_ref, b_ref, o_ref):
    h2 = a_ref[...] + c_ref[...]
    gids = jax.lax.broadcasted_iota(jnp.int32, (_G, n), 0)
    mask = (jnp.broadcast_to(bat_ref[...], (_G, n)) == gids).astype(jnp.float32)
    s = jnp.dot(mask, h2, preferred_element_type=jnp.float32)
    cnt = jnp.sum(mask, axis=1, keepdims=True)
    pooled = s / jnp.maximum(cnt, 1.0)
    z = jnp.dot(pooled, w_ref[...], preferred_element_type=jnp.float32) + b_ref[...]
    zmax = jnp.max(z, axis=1, keepdims=True)
    ez = jnp.exp(z - zmax)
    o_ref[...] = (z - zmax) - jnp.log(jnp.sum(ez, axis=1, keepdims=True))

  return pl.pallas_call(
      body,
      in_specs=[
          pl.BlockSpec((n, h), lambda: (0, 0)),
          pl.BlockSpec((n, h), lambda: (0, 0)),
          pl.BlockSpec((1, n), lambda: (0, 0)),
          pl.BlockSpec((h, c), lambda: (0, 0)),
          pl.BlockSpec((1, c), lambda: (0, 0)),
      ],
      out_specs=pl.BlockSpec((_G, c), lambda: (0, 0)),
      out_shape=jax.ShapeDtypeStruct((_G, c), jnp.float32),
  )(p0, p1, batch2, W3, b3.reshape(1, c))


def kernel(x, edge_index, batch, w_mul, W1, b1, W2, b2, W3, b3):
  n, f = x.shape
  hdim = W1.shape[1]
  e = edge_index.shape[1]

  unit = _NC * _NS * _GB
  ep = ((e + unit - 1) // unit) * unit
  pad = ep - e
  src = jnp.concatenate([edge_index[0], jnp.zeros((pad,), jnp.int32)])
  dst = jnp.concatenate([edge_index[1], jnp.zeros((pad,), jnp.int32)])
  w = jnp.concatenate([w_mul, jnp.zeros((pad,), jnp.float32)])
  src2 = src.reshape(ep // _IDXW, _IDXW)
  dst2 = dst.reshape(ep // _IDXW, _IDXW)

  np_ = ((n + 127) // 128) * 128

  h1 = _mm_bias(x, W1, b1, np_)
  p1 = _sc_aggregate(h1, src2, dst2, w, np_, hdim, ep)
  h2 = _sum_relu_mm_bias(p1[0], p1[1], W2, b2)
  p2 = _sc_aggregate(h2, src2, dst2, w, np_, hdim, ep)
  return _pool_head(p2[0, :n], p2[1, :n], batch.reshape(1, n), W3, b3)


# speedup vs baseline: 5.1616x; 1.4497x over previous
import functools

import jax
import jax.numpy as jnp
from jax import lax
from jax.experimental import pallas as pl
from jax.experimental.pallas import tpu as pltpu
from jax.experimental.pallas import tpu_sc as plsc

_NC = 2
_NS = 16
_L = 16

_GB = 512
_IDXW = 128
_G = 8


def _splat(vec16, e):
  dnums = lax.GatherDimensionNumbers(
      offset_dims=(), collapsed_slice_dims=(0,), start_index_map=(0,))
  idx = jnp.full((_L, 1), e, dtype=jnp.int32)
  return lax.gather(vec16, idx, dnums, (1,),
                    mode=lax.GatherScatterMode.PROMISE_IN_BOUNDS)


def _sc_aggregate(h, src2, dst2, w, n_nodes, n_feat, ep):
  assert n_feat % _L == 0
  ew = ep // (_NC * _NS)
  nb = ew // _GB
  assert ew % _GB == 0
  rows_ps = n_nodes // _NS
  assert n_nodes % _NS == 0
  nj = _GB // _IDXW

  mesh = plsc.VectorSubcoreMesh(core_axis_name="c", subcore_axis_name="s")

  nrows_idx = ew // _IDXW

  @functools.partial(
      pl.kernel,
      mesh=mesh,
      compiler_params=pltpu.CompilerParams(use_tc_tiling_on_sc=False),
      out_type=jax.ShapeDtypeStruct((_NC, n_nodes, n_feat), jnp.float32),
      scratch_types=[
          pltpu.VMEM((nrows_idx, _IDXW), jnp.int32),
          pltpu.VMEM((nrows_idx, _IDXW), jnp.int32),
          pltpu.VMEM((ew,), jnp.float32),
          pltpu.VMEM((_GB, n_feat), jnp.float32),
          pltpu.VMEM((_GB, n_feat), jnp.float32),
          pltpu.VMEM_SHARED((n_nodes, n_feat), jnp.float32),
          pltpu.SemaphoreType.DMA,
          pltpu.SemaphoreType.DMA,
          pltpu.SemaphoreType.DMA,
          pltpu.SemaphoreType.DMA,
      ],
  )
  def agg(h_hbm, src_hbm, dst_hbm, w_hbm, out_hbm,
          idx_v, dstidx_v, w_v, rows0_v, rows1_v, acc_sh,
          gsem0, gsem1, ssem0, ssem1):
    cid = lax.axis_index("c")
    sid = lax.axis_index("s")
    wid = cid * _NS + sid
    rows = (rows0_v, rows1_v)
    gsem = (gsem0, gsem1)
    ssem = (ssem0, ssem1)

    def zero_row(i, _):
      for k in range(n_feat // _L):
        rows0_v[i, pl.ds(k * _L, _L)] = jnp.zeros((_L,), jnp.float32)
      return _
    lax.fori_loop(0, _GB, zero_row, None)
    r0 = sid * rows_ps
    left = rows_ps
    coff = 0
    while left > 0:
      cn = min(left, _GB)
      pltpu.sync_copy(rows0_v.at[pl.ds(0, cn)], acc_sh.at[pl.ds(r0 + coff, cn)])
      coff += cn
      left -= cn

    pltpu.sync_copy(src_hbm.at[pl.ds(wid * nrows_idx, nrows_idx)], idx_v)
    pltpu.sync_copy(dst_hbm.at[pl.ds(wid * nrows_idx, nrows_idx)], dstidx_v)
    pltpu.sync_copy(w_hbm.at[pl.ds(wid * ew, ew)], w_v)
    plsc.subcore_barrier()

    def fire_gather(b):
      buf = b % 2
      return [
          pltpu.async_copy(h_hbm.at[idx_v.at[b * nj + j]],
                           rows[buf].at[pl.ds(j * _IDXW, _IDXW)], gsem[buf])
          for j in range(nj)
      ]

    def fire_scatter(b):
      buf = b % 2
      return [
          pltpu.async_copy(rows[buf].at[pl.ds(j * _IDXW, _IDXW)],
                           acc_sh.at[dstidx_v.at[b * nj + j]], ssem[buf],
                           add=True)
          for j in range(nj)
      ]

    def scale(b):
      buf = b % 2
      rv = rows[buf]

      def scale_group(g, _):
        wv = w_v[pl.ds(b * _GB + g * _L, _L)]
        for e in range(_L):
          we = _splat(wv, e)
          for k in range(n_feat // _L):
            rv[g * _L + e, pl.ds(k * _L, _L)] = (
                rv[g * _L + e, pl.ds(k * _L, _L)] * we)
        return _
      lax.fori_loop(0, _GB // _L, scale_group, None)

    gps = fire_gather(0)
    sps = [None, None]
    for b in range(nb):
      buf = b % 2
      nxt = 1 - buf
      if b + 1 < nb:
        if sps[nxt] is not None:
          for cpy in sps[nxt]:
            cpy.wait()
        gnext = fire_gather(b + 1)
      for cpy in gps:
        cpy.wait()
      scale(b)
      sps[buf] = fire_scatter(b)
      if b + 1 < nb:
        gps = gnext
    for p in sps:
      if p is not None:
        for cpy in p:
          cpy.wait()

    plsc.subcore_barrier()
    pltpu.sync_copy(acc_sh.at[pl.ds(r0, rows_ps)],
                    out_hbm.at[cid, pl.ds(r0, rows_ps)])

  return agg(h, src2, dst2, w)


def _mm_bias(x, W, b, n_out):
  f = x.shape[1]
  h = W.shape[1]
  n = n_out
  br = 1024
  grid = (n + br - 1) // br

  def body(x_ref, w_ref, b_ref, o_ref):
    o_ref[...] = (
        jnp.dot(x_ref[...], w_ref[...], preferred_element_type=jnp.float32)
        + b_ref[...])

  return pl.pallas_call(
      body,
      grid=(grid,),
      in_specs=[
          pl.BlockSpec((br, f), lambda i: (i, 0)),
          pl.BlockSpec((f, h), lambda i: (0, 0)),
          pl.BlockSpec((1, h), lambda i: (0, 0)),
      ],
      out_specs=pl.BlockSpec((br, h), lambda i: (i, 0)),
      out_shape=jax.ShapeDtypeStruct((n, h), jnp.float32),
  )(x, W, b.reshape(1, h))


def _sum_relu_mm_bias(p0, p1, W, b):
  n, f = p0.shape
  h = W.shape[1]
  br = 1024
  grid = (n + br - 1) // br

  def body(a_ref, c_ref, w_ref, b_ref, o_ref):
    s = jnp.maximum(a_ref[...] + c_ref[...], 0.0)
    o_ref[...] = (
        jnp.dot(s, w_ref[...], preferred_element_type=jnp.float32)
        + b_ref[...])

  return pl.pallas_call(
      body,
      grid=(grid,),
      in_specs=[
          pl.BlockSpec((br, f), lambda i: (i, 0)),
          pl.BlockSpec((br, f), lambda i: (i, 0)),
          pl.BlockSpec((f, h), lambda i: (0, 0)),
          pl.BlockSpec((1, h), lambda i: (0, 0)),
      ],
      out_specs=pl.BlockSpec((br, h), lambda i: (i, 0)),
      out_shape=jax.ShapeDtypeStruct((n, h), jnp.float32),
  )(p0, p1, W, b.reshape(1, h))


def _pool_head(p0, p1, batch2, W3, b3):
  n, h = p0.shape
  c = W3.shape[1]

  def body(a_ref, c_ref, bat_ref, w_ref, b_ref, o_ref):
    h2 = a_ref[...] + c_ref[...]
    gids = jax.lax.broadcasted_iota(jnp.int32, (_G, n), 0)
    mask = (jnp.broadcast_to(bat_ref[...], (_G, n)) == gids).astype(jnp.float32)
    s = jnp.dot(mask, h2, preferred_element_type=jnp.float32)
    cnt = jnp.sum(mask, axis=1, keepdims=True)
    pooled = s / jnp.maximum(cnt, 1.0)
    z = jnp.dot(pooled, w_ref[...], preferred_element_type=jnp.float32) + b_ref[...]
    zmax = jnp.max(z, axis=1, keepdims=True)
    ez = jnp.exp(z - zmax)
    o_ref[...] = (z - zmax) - jnp.log(jnp.sum(ez, axis=1, keepdims=True))

  return pl.pallas_call(
      body,
      in_specs=[
          pl.BlockSpec((n, h), lambda: (0, 0)),
          pl.BlockSpec((n, h), lambda: (0, 0)),
          pl.BlockSpec((1, n), lambda: (0, 0)),
          pl.BlockSpec((h, c), lambda: (0, 0)),
          pl.BlockSpec((1, c), lambda: (0, 0)),
      ],
      out_specs=pl.BlockSpec((_G, c), lambda: (0, 0)),
      out_shape=jax.ShapeDtypeStruct((_G, c), jnp.float32),
  )(p0, p1, batch2, W3, b3.reshape(1, c))


def kernel(x, edge_index, batch, w_mul, W1, b1, W2, b2, W3, b3):
  n, f = x.shape
  hdim = W1.shape[1]
  e = edge_index.shape[1]

  unit = _NC * _NS * _GB
  ep = ((e + unit - 1) // unit) * unit
  pad = ep - e
  src = jnp.concatenate([edge_index[0], jnp.zeros((pad,), jnp.int32)])
  dst = jnp.concatenate([edge_index[1], jnp.zeros((pad,), jnp.int32)])
  w = jnp.concatenate([w_mul, jnp.zeros((pad,), jnp.float32)])
  src2 = src.reshape(ep // _IDXW, _IDXW)
  dst2 = dst.reshape(ep // _IDXW, _IDXW)

  np_ = ((n + 127) // 128) * 128

  h1 = _mm_bias(x, W1, b1, np_)
  p1 = _sc_aggregate(h1, src2, dst2, w, np_, hdim, ep)
  h2 = _sum_relu_mm_bias(p1[0], p1[1], W2, b2)
  p2 = _sc_aggregate(h2, src2, dst2, w, np_, hdim, ep)
  return _pool_head(p2[0, :n], p2[1, :n], batch.reshape(1, n), W3, b3)


# speedup vs baseline: 8.7813x; 1.7013x over previous
import functools

import jax
import jax.numpy as jnp
from jax import lax
from jax.experimental import pallas as pl
from jax.experimental.pallas import tpu as pltpu
from jax.experimental.pallas import tpu_sc as plsc

_NC = 2
_NS = 16
_L = 16

_GB = 512
_IDXW = 128
_G = 8


def _splat(vec16, e):
  dnums = lax.GatherDimensionNumbers(
      offset_dims=(), collapsed_slice_dims=(0,), start_index_map=(0,))
  idx = jnp.full((_L, 1), e, dtype=jnp.int32)
  return lax.gather(vec16, idx, dnums, (1,),
                    mode=lax.GatherScatterMode.PROMISE_IN_BOUNDS)


def _sc_aggregate(h2c, src2, dst2, w, n_nodes, n_feat, ep):
  hf = n_feat // _NC
  assert hf % _L == 0
  et = ep // _NS
  nb = et // _GB
  assert et % _GB == 0
  rows_ps = n_nodes // _NS
  assert n_nodes % _NS == 0 and rows_ps % 8 == 0
  nj = _GB // _IDXW
  nrows_idx = et // _IDXW

  mesh = plsc.VectorSubcoreMesh(core_axis_name="c", subcore_axis_name="s")

  @functools.partial(
      pl.kernel,
      mesh=mesh,
      compiler_params=pltpu.CompilerParams(use_tc_tiling_on_sc=False),
      out_type=jax.ShapeDtypeStruct((_NC, n_nodes, hf), jnp.float32),
      scratch_types=[
          pltpu.VMEM((nrows_idx, _IDXW), jnp.int32),
          pltpu.VMEM((nrows_idx, _IDXW), jnp.int32),
          pltpu.VMEM((et,), jnp.float32),
          pltpu.VMEM((_GB, hf), jnp.float32),
          pltpu.VMEM((_GB, hf), jnp.float32),
          pltpu.VMEM_SHARED((n_nodes, hf), jnp.float32),
          pltpu.VMEM_SHARED((n_nodes, hf), jnp.float32),
          pltpu.SemaphoreType.DMA,
          pltpu.SemaphoreType.DMA,
          pltpu.SemaphoreType.DMA,
          pltpu.SemaphoreType.DMA,
      ],
  )
  def agg(h_hbm, src_hbm, dst_hbm, w_hbm, out_hbm,
          idx_v, dstidx_v, w_v, rows0_v, rows1_v, acc_sh, tbl_sh,
          gsem0, gsem1, ssem0, ssem1):
    cid = lax.axis_index("c")
    sid = lax.axis_index("s")
    rows = (rows0_v, rows1_v)
    gsem = (gsem0, gsem1)
    ssem = (ssem0, ssem1)

    def zero_row(i, _):
      for k in range(hf // _L):
        rows0_v[i, pl.ds(k * _L, _L)] = jnp.zeros((_L,), jnp.float32)
      return _
    lax.fori_loop(0, _GB, zero_row, None)
    r0 = sid * rows_ps
    left = rows_ps
    coff = 0
    while left > 0:
      cn = min(left, _GB)
      pltpu.sync_copy(rows0_v.at[pl.ds(0, cn)], acc_sh.at[pl.ds(r0 + coff, cn)])
      coff += cn
      left -= cn

    pltpu.sync_copy(h_hbm.at[cid, pl.ds(r0, rows_ps)],
                    tbl_sh.at[pl.ds(r0, rows_ps)])
    pltpu.sync_copy(src_hbm.at[pl.ds(sid * nrows_idx, nrows_idx)], idx_v)
    pltpu.sync_copy(dst_hbm.at[pl.ds(sid * nrows_idx, nrows_idx)], dstidx_v)
    pltpu.sync_copy(w_hbm.at[pl.ds(sid * et, et)], w_v)
    plsc.subcore_barrier()

    def fire_gather(b):
      buf = b % 2
      return [
          pltpu.async_copy(tbl_sh.at[idx_v.at[b * nj + j]],
                           rows[buf].at[pl.ds(j * _IDXW, _IDXW)], gsem[buf])
          for j in range(nj)
      ]

    def fire_scatter(b):
      buf = b % 2
      return [
          pltpu.async_copy(rows[buf].at[pl.ds(j * _IDXW, _IDXW)],
                           acc_sh.at[dstidx_v.at[b * nj + j]], ssem[buf],
                           add=True)
          for j in range(nj)
      ]

    def scale(b):
      buf = b % 2
      rv = rows[buf]

      def scale_group(g, _):
        wv = w_v[pl.ds(b * _GB + g * _L, _L)]
        for e in range(_L):
          we = _splat(wv, e)
          for k in range(hf // _L):
            rv[g * _L + e, pl.ds(k * _L, _L)] = (
                rv[g * _L + e, pl.ds(k * _L, _L)] * we)
        return _
      lax.fori_loop(0, _GB // _L, scale_group, None)

    gps = fire_gather(0)
    sps = [None, None]
    for b in range(nb):
      buf = b % 2
      nxt = 1 - buf
      if b + 1 < nb:
        if sps[nxt] is not None:
          for cpy in sps[nxt]:
            cpy.wait()
        gnext = fire_gather(b + 1)
      for cpy in gps:
        cpy.wait()
      scale(b)
      sps[buf] = fire_scatter(b)
      if b + 1 < nb:
        gps = gnext
    for p in sps:
      if p is not None:
        for cpy in p:
          cpy.wait()

    plsc.subcore_barrier()
    pltpu.sync_copy(acc_sh.at[pl.ds(r0, rows_ps)],
                    out_hbm.at[cid, pl.ds(r0, rows_ps)])

  return agg(h2c, src2, dst2, w)


def _mm_bias_split(x, Wc, bc, n_out):
  f = x.shape[1]
  hf = Wc.shape[2]
  br = 1024
  grid = ((n_out + br - 1) // br, 2)

  def body(x_ref, w_ref, b_ref, o_ref):
    o_ref[0] = (
        jnp.dot(x_ref[...], w_ref[0], preferred_element_type=jnp.float32)
        + b_ref[0])

  return pl.pallas_call(
      body,
      grid=grid,
      in_specs=[
          pl.BlockSpec((br, f), lambda i, j: (i, 0)),
          pl.BlockSpec((1, f, hf), lambda i, j: (j, 0, 0)),
          pl.BlockSpec((1, 1, hf), lambda i, j: (j, 0, 0)),
      ],
      out_specs=pl.BlockSpec((1, br, hf), lambda i, j: (j, i, 0)),
      out_shape=jax.ShapeDtypeStruct((2, n_out, hf), jnp.float32),
  )(x, Wc, bc)


def _relu_mm_bias_split(p, Wc, bc):
  n = p.shape[1]
  hf = p.shape[2]
  br = 1024
  grid = ((n + br - 1) // br, 2)

  def body(p_ref, w_ref, b_ref, o_ref):
    s = jnp.maximum(jnp.concatenate([p_ref[0], p_ref[1]], axis=1), 0.0)
    o_ref[0] = (
        jnp.dot(s, w_ref[0], preferred_element_type=jnp.float32) + b_ref[0])

  return pl.pallas_call(
      body,
      grid=grid,
      in_specs=[
          pl.BlockSpec((2, br, hf), lambda i, j: (0, i, 0)),
          pl.BlockSpec((1, 2 * hf, hf), lambda i, j: (j, 0, 0)),
          pl.BlockSpec((1, 1, hf), lambda i, j: (j, 0, 0)),
      ],
      out_specs=pl.BlockSpec((1, br, hf), lambda i, j: (j, i, 0)),
      out_shape=jax.ShapeDtypeStruct((2, n, hf), jnp.float32),
  )(p, Wc, bc)


def _pool_head(p, batch2, W3, b3):
  n = p.shape[1]
  hf = p.shape[2]
  c = W3.shape[1]

  def body(p_ref, bat_ref, w_ref, b_ref, o_ref):
    h2 = jnp.concatenate([p_ref[0], p_ref[1]], axis=1)
    gids = jax.lax.broadcasted_iota(jnp.int32, (_G, n), 0)
    mask = (jnp.broadcast_to(bat_ref[...], (_G, n)) == gids).astype(jnp.float32)
    s = jnp.dot(mask, h2, preferred_element_type=jnp.float32)
    cnt = jnp.sum(mask, axis=1, keepdims=True)
    pooled = s / jnp.maximum(cnt, 1.0)
    z = jnp.dot(pooled, w_ref[...], preferred_element_type=jnp.float32) + b_ref[...]
    zmax = jnp.max(z, axis=1, keepdims=True)
    ez = jnp.exp(z - zmax)
    o_ref[...] = (z - zmax) - jnp.log(jnp.sum(ez, axis=1, keepdims=True))

  return pl.pallas_call(
      body,
      in_specs=[
          pl.BlockSpec((2, n, hf), lambda: (0, 0, 0)),
          pl.BlockSpec((1, n), lambda: (0, 0)),
          pl.BlockSpec((2 * hf, c), lambda: (0, 0)),
          pl.BlockSpec((1, c), lambda: (0, 0)),
      ],
      out_specs=pl.BlockSpec((_G, c), lambda: (0, 0)),
      out_shape=jax.ShapeDtypeStruct((_G, c), jnp.float32),
  )(p, batch2, W3, b3.reshape(1, c))


def kernel(x, edge_index, batch, w_mul, W1, b1, W2, b2, W3, b3):
  n, f = x.shape
  hdim = W1.shape[1]
  hf = hdim // _NC
  e = edge_index.shape[1]

  unit = _NS * _GB
  ep = ((e + unit - 1) // unit) * unit
  pad = ep - e
  spread = (jnp.arange(pad, dtype=jnp.int32) * 61) % n
  src = jnp.concatenate([edge_index[0], spread])
  dst = jnp.concatenate([edge_index[1], spread])
  w = jnp.concatenate([w_mul, jnp.zeros((pad,), jnp.float32)])
  src2 = src.reshape(ep // _IDXW, _IDXW)
  dst2 = dst.reshape(ep // _IDXW, _IDXW)

  np_ = ((n + 127) // 128) * 128

  W1c = W1.reshape(f, _NC, hf).transpose(1, 0, 2)
  b1c = b1.reshape(1, _NC, hf).transpose(1, 0, 2)
  W2c = W2.reshape(hdim, _NC, hf).transpose(1, 0, 2)
  b2c = b2.reshape(1, _NC, hf).transpose(1, 0, 2)

  h1 = _mm_bias_split(x, W1c, b1c, np_)
  p1 = _sc_aggregate(h1, src2, dst2, w, np_, hdim, ep)
  h2 = _relu_mm_bias_split(p1, W2c, b2c)
  p2 = _sc_aggregate(h2, src2, dst2, w, np_, hdim, ep)
  return _pool_head(p2[:, :n], batch.reshape(1, n), W3, b3)


# speedup vs baseline: 10.1627x; 1.1573x over previous
import functools

import jax
import jax.numpy as jnp
from jax import lax
from jax.experimental import pallas as pl
from jax.experimental.pallas import tpu as pltpu
from jax.experimental.pallas import tpu_sc as plsc

_NC = 2
_NS = 16
_L = 16

_GB = 640
_IDXW = 128
_G = 8


def _splat(vec16, e):
  dnums = lax.GatherDimensionNumbers(
      offset_dims=(), collapsed_slice_dims=(0,), start_index_map=(0,))
  idx = jnp.full((_L, 1), e, dtype=jnp.int32)
  return lax.gather(vec16, idx, dnums, (1,),
                    mode=lax.GatherScatterMode.PROMISE_IN_BOUNDS)


def _sc_pass(tbl, pk2, w, lo16, n_nodes, n_feat, ep, phase1):
  hf = n_feat // _NC
  assert hf == 2 * _L
  et = ep // _NS
  nb = et // _GB
  assert et % _GB == 0
  rows_ps = n_nodes // _NS
  assert n_nodes % _NS == 0 and rows_ps % 8 == 0 and rows_ps <= _GB
  nj = _GB // _IDXW
  nrows_idx = et // _IDXW

  mesh = plsc.VectorSubcoreMesh(core_axis_name="c", subcore_axis_name="s")

  if phase1:
    out_type = (
        jax.ShapeDtypeStruct((_NC, n_nodes, hf), jnp.float32),
        jax.ShapeDtypeStruct((_NC, _NS, _G * _L), jnp.float32),
    )
    extra_scratch = [
        pltpu.VMEM((_L,), jnp.int32),
        pltpu.VMEM((_G * _L,), jnp.float32),
    ]
  else:
    out_type = jax.ShapeDtypeStruct((_NC, n_nodes, hf), jnp.float32)
    extra_scratch = []
  tbl_dtype = jnp.float32

  @functools.partial(
      pl.kernel,
      mesh=mesh,
      compiler_params=pltpu.CompilerParams(use_tc_tiling_on_sc=False),
      out_type=out_type,
      scratch_types=[
          pltpu.VMEM((nrows_idx, _IDXW), jnp.int32),
          pltpu.VMEM((nrows_idx, _IDXW), jnp.int32),
          pltpu.VMEM((et,), jnp.float32),
          pltpu.VMEM((_GB, hf), jnp.float32),
          pltpu.VMEM((_GB, hf), jnp.float32),
          pltpu.VMEM_SHARED((n_nodes, hf), tbl_dtype),
          pltpu.VMEM_SHARED((n_nodes, hf), jnp.float32),
          pltpu.SemaphoreType.DMA,
          pltpu.SemaphoreType.DMA,
          pltpu.SemaphoreType.DMA,
          pltpu.SemaphoreType.DMA,
      ] + extra_scratch,
  )
  def agg(*args):
    if phase1:
      (tbl_hbm, pk_hbm, w_hbm, lo_hbm, out_hbm, bins_hbm,
       idx_v, dstidx_v, w_v, msg0_v, msg1_v, tbl_sh, acc_sh,
       gsem0, gsem1, ssem0, ssem1, lo_v, bins_v) = args
    else:
      (tbl_hbm, pk_hbm, w_hbm, out_hbm,
       idx_v, dstidx_v, w_v, msg0_v, msg1_v, tbl_sh, acc_sh,
       gsem0, gsem1, ssem0, ssem1) = args
    msg = (msg0_v, msg1_v)
    gbufs = msg
    gsem = (gsem0, gsem1)
    ssem = (ssem0, ssem1)
    cid = lax.axis_index("c")
    sid = lax.axis_index("s")
    r0 = sid * rows_ps

    def zero_row(i, _):
      for k in range(hf // _L):
        msg0_v[i, pl.ds(k * _L, _L)] = jnp.zeros((_L,), jnp.float32)
      return _
    lax.fori_loop(0, rows_ps, zero_row, None)
    pltpu.sync_copy(msg0_v.at[pl.ds(0, rows_ps)], acc_sh.at[pl.ds(r0, rows_ps)])
    if phase1:
      pltpu.sync_copy(lo_hbm, lo_v)
    pltpu.sync_copy(tbl_hbm.at[cid, pl.ds(r0, rows_ps)],
                    tbl_sh.at[pl.ds(r0, rows_ps)])
    pltpu.sync_copy(pk_hbm.at[pl.ds(sid * nrows_idx, nrows_idx)], idx_v)
    def unpack_row(rr, _):
      for cc in range(_IDXW // _L):
        p16 = idx_v[rr, pl.ds(cc * _L, _L)]
        idx_v[rr, pl.ds(cc * _L, _L)] = jnp.bitwise_and(p16, 16383)
        dstidx_v[rr, pl.ds(cc * _L, _L)] = lax.shift_right_logical(p16, 14)
      return _
    lax.fori_loop(0, nrows_idx, unpack_row, None)
    pltpu.sync_copy(w_hbm.at[pl.ds(sid * et, et)], w_v)
    plsc.subcore_barrier()

    def fire_gather(b):
      buf = b % 2
      return [
          pltpu.async_copy(tbl_sh.at[idx_v.at[b * nj + j]],
                           gbufs[buf].at[pl.ds(j * _IDXW, _IDXW)], gsem[buf])
          for j in range(nj)
      ]

    def fire_scatter(b):
      buf = b % 2
      return [
          pltpu.async_copy(msg[buf].at[pl.ds(j * _IDXW, _IDXW)],
                           acc_sh.at[dstidx_v.at[b * nj + j]], ssem[buf],
                           add=True)
          for j in range(nj)
      ]

    def scale(b, bins_state):
      buf = b % 2
      mv = msg[buf]
      if phase1:
        lov = lo_v[...]
        los = [_splat(lov, k) for k in range(_G)]
        ups = [_splat(lov, _G + k) for k in range(_G)]

      def scale_group(g, carry):
        wv = w_v[pl.ds(b * _GB + g * _L, _L)]
        if phase1:
          frow = b * (_GB // _IDXW) + g // (_IDXW // _L)
          fcol = (g % (_IDXW // _L)) * _L
          d16 = dstidx_v[frow, pl.ds(fcol, _L)]
          zero = jnp.zeros((_L,), jnp.float32)
          carry = tuple(
              carry[k] + jnp.where((d16 >= los[k]) & (d16 < ups[k]), wv, zero)
              for k in range(_G))
        for e in range(_L):
          we = _splat(wv, e)
          r = g * _L + e
          for k in range(hf // _L):
            v = mv[r, pl.ds(k * _L, _L)]
            if not phase1:
              v = jnp.maximum(v, 0.0)
            mv[r, pl.ds(k * _L, _L)] = v * we
        return carry
      return lax.fori_loop(0, _GB // _L, scale_group, bins_state)

    bins_state = tuple(jnp.zeros((_L,), jnp.float32) for _ in range(_G))
    gps = fire_gather(0)
    sps = [None, None]
    for b in range(nb):
      buf = b % 2
      nxt = 1 - buf
      if b + 1 < nb:
        if sps[nxt] is not None:
          for cpy in sps[nxt]:
            cpy.wait()
        gnext = fire_gather(b + 1)
      for cpy in gps:
        cpy.wait()
      bins_state = scale(b, bins_state)
      sps[buf] = fire_scatter(b)
      if b + 1 < nb:
        gps = gnext
    for p in sps:
      if p is not None:
        for cpy in p:
          cpy.wait()

    plsc.subcore_barrier()
    pltpu.sync_copy(acc_sh.at[pl.ds(r0, rows_ps)],
                    out_hbm.at[cid, pl.ds(r0, rows_ps)])
    if phase1:
      for k in range(_G):
        bins_v[pl.ds(k * _L, _L)] = bins_state[k]
      pltpu.sync_copy(bins_v, bins_hbm.at[cid, sid])

  if phase1:
    return agg(tbl, pk2, w, lo16)
  return agg(tbl, pk2, w)


def _mm_bias_split(x, Wc, bc, n_out):
  f = x.shape[1]
  hf = Wc.shape[2]
  br = 1024
  grid = ((n_out + br - 1) // br, 2)

  def body(x_ref, w_ref, b_ref, o_ref):
    o_ref[0] = (
        jnp.dot(x_ref[...], w_ref[0], preferred_element_type=jnp.float32)
        + b_ref[0])

  return pl.pallas_call(
      body,
      grid=grid,
      in_specs=[
          pl.BlockSpec((br, f), lambda i, j: (i, 0)),
          pl.BlockSpec((1, f, hf), lambda i, j: (j, 0, 0)),
          pl.BlockSpec((1, 1, hf), lambda i, j: (j, 0, 0)),
      ],
      out_specs=pl.BlockSpec((1, br, hf), lambda i, j: (j, i, 0)),
      out_shape=jax.ShapeDtypeStruct((2, n_out, hf), jnp.float32),
  )(x, Wc, bc)


def _pool_head(p, bins3, batch2, W2, b2, W3, b3):
  n = p.shape[1]
  hf = p.shape[2]
  c = W3.shape[1]
  nbin = bins3.shape[0]

  def body(p_ref, bins_ref, bat_ref, w2_ref, b2_ref, w3_ref, b3_ref, o_ref):
    h2 = jnp.concatenate([p_ref[0], p_ref[1]], axis=1)
    gids = jax.lax.broadcasted_iota(jnp.int32, (_G, n), 0)
    mask = (jnp.broadcast_to(bat_ref[...], (_G, n)) == gids).astype(jnp.float32)
    s = jnp.dot(mask, h2, preferred_element_type=jnp.float32)
    cnt = jnp.maximum(jnp.sum(mask, axis=1, keepdims=True), 1.0)
    deg8 = jnp.sum(bins_ref[...], axis=(0, 2)).reshape(_G, 1)
    pooled = (jnp.dot(s / cnt, w2_ref[...], preferred_element_type=jnp.float32)
              + (deg8 / cnt) * b2_ref[...])
    z = (jnp.dot(pooled, w3_ref[...], preferred_element_type=jnp.float32)
         + b3_ref[...])
    zmax = jnp.max(z, axis=1, keepdims=True)
    ez = jnp.exp(z - zmax)
    o_ref[...] = (z - zmax) - jnp.log(jnp.sum(ez, axis=1, keepdims=True))

  return pl.pallas_call(
      body,
      in_specs=[
          pl.BlockSpec((2, n, hf), lambda: (0, 0, 0)),
          pl.BlockSpec((nbin, _G, _L), lambda: (0, 0, 0)),
          pl.BlockSpec((1, n), lambda: (0, 0)),
          pl.BlockSpec((2 * hf, 2 * hf), lambda: (0, 0)),
          pl.BlockSpec((1, 2 * hf), lambda: (0, 0)),
          pl.BlockSpec((2 * hf, c), lambda: (0, 0)),
          pl.BlockSpec((1, c), lambda: (0, 0)),
      ],
      out_specs=pl.BlockSpec((_G, c), lambda: (0, 0)),
      out_shape=jax.ShapeDtypeStruct((_G, c), jnp.float32),
  )(p, bins3, batch2, W2, b2.reshape(1, 2 * hf), W3, b3.reshape(1, c))


def kernel(x, edge_index, batch, w_mul, W1, b1, W2, b2, W3, b3):
  n, f = x.shape
  hdim = W1.shape[1]
  hf = hdim // _NC
  e = edge_index.shape[1]

  unit = _NS * _GB
  ep = ((e + unit - 1) // unit) * unit
  pad = ep - e
  spread = (jnp.arange(pad, dtype=jnp.int32) * 61) % n
  src = jnp.concatenate([edge_index[0], spread])
  dst = jnp.concatenate([edge_index[1], spread])
  w = jnp.concatenate([w_mul, jnp.zeros((pad,), jnp.float32)])
  pk2 = (src | (dst << 14)).reshape(ep // _IDXW, _IDXW)

  np_ = ((n + 127) // 128) * 128

  bounds = jnp.searchsorted(batch, jnp.arange(1, _G + 1, dtype=jnp.int32)
                            ).astype(jnp.int32)
  lows = jnp.concatenate([jnp.zeros((1,), jnp.int32), bounds[:_G - 1]])
  lo16 = jnp.concatenate([lows, bounds])

  W1c = W1.reshape(f, _NC, hf).transpose(1, 0, 2)
  b1c = b1.reshape(1, _NC, hf).transpose(1, 0, 2)

  h1 = _mm_bias_split(x, W1c, b1c, np_)
  p1, bins = _sc_pass(h1, pk2, w, lo16, np_, hdim, ep, phase1=True)
  p2 = _sc_pass(p1, pk2, w, None, np_, hdim, ep, phase1=False)
  bins3 = bins.reshape(_NC * _NS, _G, _L)
  return _pool_head(p2[:, :n], bins3, batch.reshape(1, n), W2, b2, W3, b3)
